# baseline (speedup 1.0000x reference)
import functools

import jax
import jax.numpy as jnp
from jax import lax
from jax.experimental import pallas as pl
from jax.experimental.pallas import tpu as pltpu
from jax.experimental.pallas import tpu_sc as plsc

VOCAB = 1000000
EMB = 64
N = 100000

NC = 2
NS = 16
NW = NC * NS
ROWS_PER_W = 3200
NCHUNK = 5
CH = 640
N_PAD = NW * ROWS_PER_W

TC_TILE = 2048


def _sc_gather(U, V, ui, vi):
    mesh = plsc.VectorSubcoreMesh(
        core_axis_name="c", subcore_axis_name="s",
        num_cores=NC, num_subcores=NS,
    )

    @functools.partial(
        pl.kernel,
        out_type=(
            jax.ShapeDtypeStruct((N_PAD, EMB), jnp.float32),
            jax.ShapeDtypeStruct((N_PAD, EMB), jnp.float32),
        ),
        mesh=mesh,
        scratch_types=[
            pltpu.VMEM((CH,), jnp.int32),
            pltpu.VMEM((CH, EMB), jnp.float32),
            pltpu.SemaphoreType.DMA,
        ],
        compiler_params=pltpu.CompilerParams(use_tc_tiling_on_sc=False),
    )
    def k(u_hbm, v_hbm, ui_hbm, vi_hbm, ug_hbm, vg_hbm, idx_v, rows_v, sem):
        wid = lax.axis_index("s") * NC + lax.axis_index("c")
        base = wid * ROWS_PER_W
        for tbl_hbm, i_hbm, o_hbm in ((u_hbm, ui_hbm, ug_hbm),
                                      (v_hbm, vi_hbm, vg_hbm)):
            for ci in range(NCHUNK):
                pltpu.sync_copy(i_hbm.at[pl.ds(base + ci * CH, CH)], idx_v)
                pltpu.async_copy(tbl_hbm.at[idx_v], rows_v, sem).wait()
                pltpu.sync_copy(rows_v, o_hbm.at[pl.ds(base + ci * CH, CH)])

    return k(U, V, ui, vi)


def _tc_bilinear(UG, VG, B00, bu, bv, c11):
    grid = N_PAD // TC_TILE

    def body(ug_ref, vg_ref, b00_ref, bu_ref, bv_ref, c_ref, out_ref):
        u = ug_ref[...]
        v = vg_ref[...]
        cu = jnp.dot(u, b00_ref[...], preferred_element_type=jnp.float32)
        t = jnp.sum((cu + bv_ref[...]) * v, axis=1)
        t2 = jnp.dot(u, bu_ref[...], preferred_element_type=jnp.float32)[:, 0]
        out_ref[...] = t + t2 + c_ref[0, 0]

    return pl.pallas_call(
        body,
        grid=(grid,),
        in_specs=[
            pl.BlockSpec((TC_TILE, EMB), lambda i: (i, 0)),
            pl.BlockSpec((TC_TILE, EMB), lambda i: (i, 0)),
            pl.BlockSpec((EMB, EMB), lambda i: (0, 0)),
            pl.BlockSpec((EMB, 1), lambda i: (0, 0)),
            pl.BlockSpec((1, EMB), lambda i: (0, 0)),
            pl.BlockSpec((1, 1), lambda i: (0, 0)),
        ],
        out_specs=pl.BlockSpec((TC_TILE,), lambda i: (i,)),
        out_shape=jax.ShapeDtypeStruct((N_PAD,), jnp.float32),
    )(UG, VG, B00, bu, bv, c11)


@jax.jit
def kernel(U, V, B, us_ind, vs_ind):
    pad = N_PAD - N
    ui = jnp.concatenate(
        [us_ind.astype(jnp.int32), jnp.zeros((pad,), jnp.int32)])
    vi = jnp.concatenate(
        [vs_ind.astype(jnp.int32), jnp.zeros((pad,), jnp.int32)])

    UG, VG = _sc_gather(U, V, ui, vi)

    B00 = B[:EMB, :EMB]
    bu = B[:EMB, EMB:]
    bv = B[EMB:, :EMB]
    c11 = B[EMB:, EMB:]
    out = _tc_bilinear(UG, VG, B00, bu, bv, c11)
    return out[:N]


# speedup vs baseline: 1.0154x; 1.0154x over previous
import functools

import jax
import jax.numpy as jnp
from jax import lax
from jax.experimental import pallas as pl
from jax.experimental.pallas import tpu as pltpu
from jax.experimental.pallas import tpu_sc as plsc

VOCAB = 1000000
EMB = 64
N = 100000

NC = 2
NS = 16
NW = NC * NS
ROWS_PER_W = 3200
CH = 320
NBUF = 4
NCHUNK = (2 * ROWS_PER_W) // CH
N_PAD = NW * ROWS_PER_W

TC_TILE = 2048


def _sc_gather(U, V, ui, vi):
    mesh = plsc.VectorSubcoreMesh(
        core_axis_name="c", subcore_axis_name="s",
        num_cores=NC, num_subcores=NS,
    )

    half = ROWS_PER_W // CH

    @functools.partial(
        pl.kernel,
        out_type=(
            jax.ShapeDtypeStruct((N_PAD, EMB), jnp.float32),
            jax.ShapeDtypeStruct((N_PAD, EMB), jnp.float32),
        ),
        mesh=mesh,
        scratch_types=[
            pltpu.VMEM((NBUF, CH), jnp.int32),
            pltpu.VMEM((NBUF, CH, EMB), jnp.float32),
            [pltpu.SemaphoreType.DMA for _ in range(NBUF)],
            [pltpu.SemaphoreType.DMA for _ in range(NBUF)],
        ],
        compiler_params=pltpu.CompilerParams(use_tc_tiling_on_sc=False),
    )
    def k(u_hbm, v_hbm, ui_hbm, vi_hbm, ug_hbm, vg_hbm,
          idx_v, rows_v, gsems, wsems):
        wid = lax.axis_index("s") * NC + lax.axis_index("c")
        base = wid * ROWS_PER_W

        def tbl(ci):
            return u_hbm if ci < half else v_hbm

        def idx_src(ci):
            src = ui_hbm if ci < half else vi_hbm
            off = base + (ci % half) * CH
            return src.at[pl.ds(off, CH)]

        def out_dst(ci):
            dst = ug_hbm if ci < half else vg_hbm
            off = base + (ci % half) * CH
            return dst.at[pl.ds(off, CH)]

        def fire(ci):
            b = ci % NBUF
            pltpu.sync_copy(idx_src(ci), idx_v.at[b])
            pltpu.async_copy(tbl(ci).at[idx_v.at[b]], rows_v.at[b], gsems[b])

        for ci in range(NBUF):
            fire(ci)
        for ci in range(NCHUNK):
            b = ci % NBUF
            pltpu.make_async_copy(
                tbl(ci).at[idx_v.at[b]], rows_v.at[b], gsems[b]).wait()
            pltpu.async_copy(rows_v.at[b], out_dst(ci), wsems[b])
            nci = ci + NBUF
            if nci < NCHUNK:
                pltpu.make_async_copy(
                    rows_v.at[b], out_dst(ci), wsems[b]).wait()
                fire(nci)
        for ci in range(NCHUNK - NBUF, NCHUNK):
            b = ci % NBUF
            pltpu.make_async_copy(
                rows_v.at[b], out_dst(ci), wsems[b]).wait()

    return k(U, V, ui, vi)


def _tc_bilinear(UG, VG, B00, bu, bv, c11):
    grid = N_PAD // TC_TILE

    def body(ug_ref, vg_ref, b00_ref, bu_ref, bv_ref, c_ref, out_ref):
        u = ug_ref[...]
        v = vg_ref[...]
        cu = jnp.dot(u, b00_ref[...], preferred_element_type=jnp.float32)
        t = jnp.sum((cu + bv_ref[...]) * v, axis=1)
        t2 = jnp.dot(u, bu_ref[...], preferred_element_type=jnp.float32)[:, 0]
        out_ref[...] = t + t2 + c_ref[0, 0]

    return pl.pallas_call(
        body,
        grid=(grid,),
        in_specs=[
            pl.BlockSpec((TC_TILE, EMB), lambda i: (i, 0)),
            pl.BlockSpec((TC_TILE, EMB), lambda i: (i, 0)),
            pl.BlockSpec((EMB, EMB), lambda i: (0, 0)),
            pl.BlockSpec((EMB, 1), lambda i: (0, 0)),
            pl.BlockSpec((1, EMB), lambda i: (0, 0)),
            pl.BlockSpec((1, 1), lambda i: (0, 0)),
        ],
        out_specs=pl.BlockSpec((TC_TILE,), lambda i: (i,)),
        out_shape=jax.ShapeDtypeStruct((N_PAD,), jnp.float32),
    )(UG, VG, B00, bu, bv, c11)


@jax.jit
def kernel(U, V, B, us_ind, vs_ind):
    pad = N_PAD - N
    ui = jnp.concatenate(
        [us_ind.astype(jnp.int32), jnp.zeros((pad,), jnp.int32)])
    vi = jnp.concatenate(
        [vs_ind.astype(jnp.int32), jnp.zeros((pad,), jnp.int32)])

    UG, VG = _sc_gather(U, V, ui, vi)

    B00 = B[:EMB, :EMB]
    bu = B[:EMB, EMB:]
    bv = B[EMB:, :EMB]
    c11 = B[EMB:, EMB:]
    out = _tc_bilinear(UG, VG, B00, bu, bv, c11)
    return out[:N]


# speedup vs baseline: 1.3634x; 1.3428x over previous
import functools

import jax
import jax.numpy as jnp
from jax import lax
from jax.experimental import pallas as pl
from jax.experimental.pallas import tpu as pltpu
from jax.experimental.pallas import tpu_sc as plsc

VOCAB = 1000000
TBW = 2048
SPLIT = TBW * 245
EMB = 64
N = 100000

NC = 2
NS = 16
NW = NC * NS
ROWS_PER_W = 3200
CH = 320
NBUF = 3
NCHUNK = ROWS_PER_W // CH
N_PAD = NW * ROWS_PER_W

TC_TILE = 2048


def _tc_repack(WT):
    grid = SPLIT // TBW
    noff = SPLIT // TBW

    def body(wt0_ref, wt1_ref, out_ref):
        x0 = wt0_ref[...]
        x1 = wt1_ref[...]
        out_ref[...] = jnp.concatenate(
            [jnp.transpose(x0), jnp.transpose(x1)], axis=1)

    return pl.pallas_call(
        body,
        grid=(grid,),
        in_specs=[
            pl.BlockSpec((EMB, TBW), lambda i: (0, i)),
            pl.BlockSpec(
                (EMB, TBW),
                lambda i: (0, jnp.minimum(i + noff, VOCAB // TBW))),
        ],
        out_specs=pl.BlockSpec((TBW, 128), lambda i: (i, 0)),
        out_shape=jax.ShapeDtypeStruct((SPLIT, 128), jnp.float32),
    )(WT, WT)


def _sc_gather(W2, idxh):
    mesh = plsc.VectorSubcoreMesh(
        core_axis_name="c", subcore_axis_name="s",
        num_cores=NC, num_subcores=NS,
    )

    @functools.partial(
        pl.kernel,
        out_type=jax.ShapeDtypeStruct((N_PAD, 128), jnp.float32),
        mesh=mesh,
        scratch_types=[
            [pltpu.VMEM((CH,), jnp.int32) for _ in range(NBUF)],
            [pltpu.VMEM((CH, 128), jnp.float32) for _ in range(NBUF)],
            [pltpu.SemaphoreType.DMA for _ in range(NBUF)],
            [pltpu.SemaphoreType.DMA for _ in range(NBUF)],
        ],
    )
    def k(w_hbm, i_hbm, o_hbm, idx_vs, rows_vs, gsems, wsems):
        wid = lax.axis_index("s") * NC + lax.axis_index("c")
        base = wid * ROWS_PER_W

        def fire(ci):
            b = ci % NBUF
            off = base + ci * CH
            pltpu.sync_copy(i_hbm.at[pl.ds(off, CH)], idx_vs[b])
            pltpu.async_copy(w_hbm.at[idx_vs[b]], rows_vs[b], gsems[b])

        for ci in range(NBUF):
            fire(ci)
        for ci in range(NCHUNK):
            b = ci % NBUF
            off = base + ci * CH
            pltpu.make_async_copy(
                w_hbm.at[idx_vs[b]], rows_vs[b], gsems[b]).wait()
            pltpu.async_copy(
                rows_vs[b], o_hbm.at[pl.ds(off, CH)], wsems[b])
            nci = ci + NBUF
            if nci < NCHUNK:
                pltpu.make_async_copy(
                    rows_vs[b], o_hbm.at[pl.ds(off, CH)], wsems[b]).wait()
                fire(nci)
        for ci in range(NCHUNK - NBUF, NCHUNK):
            b = ci % NBUF
            off = base + ci * CH
            pltpu.make_async_copy(
                rows_vs[b], o_hbm.at[pl.ds(off, CH)], wsems[b]).wait()

    return k(W2, idxh)


def _tc_bilinear(UG, VG, pu, pv, B00, bu, bv, c11):
    grid = N_PAD // TC_TILE

    def body(ug_ref, vg_ref, pu_ref, pv_ref, b00_ref, bu_ref, bv_ref, c_ref,
             out_ref):
        ug = ug_ref[...]
        vg = vg_ref[...]
        u = jnp.where(pu_ref[...] == 1, ug[:, EMB:], ug[:, :EMB])
        v = jnp.where(pv_ref[...] == 1, vg[:, EMB:], vg[:, :EMB])
        cu = jnp.dot(u, b00_ref[...], preferred_element_type=jnp.float32)
        t = jnp.sum((cu + bv_ref[...]) * v, axis=1)
        t2 = jnp.dot(u, bu_ref[...], preferred_element_type=jnp.float32)[:, 0]
        out_ref[...] = t + t2 + c_ref[0, 0]

    return pl.pallas_call(
        body,
        grid=(grid,),
        in_specs=[
            pl.BlockSpec((TC_TILE, 128), lambda i: (i, 0)),
            pl.BlockSpec((TC_TILE, 128), lambda i: (i, 0)),
            pl.BlockSpec((TC_TILE, 1), lambda i: (i, 0)),
            pl.BlockSpec((TC_TILE, 1), lambda i: (i, 0)),
            pl.BlockSpec((EMB, EMB), lambda i: (0, 0)),
            pl.BlockSpec((EMB, 1), lambda i: (0, 0)),
            pl.BlockSpec((1, EMB), lambda i: (0, 0)),
            pl.BlockSpec((1, 1), lambda i: (0, 0)),
        ],
        out_specs=pl.BlockSpec((TC_TILE,), lambda i: (i,)),
        out_shape=jax.ShapeDtypeStruct((N_PAD,), jnp.float32),
    )(UG, VG, pu, pv, B00, bu, bv, c11)


@jax.jit
def kernel(U, V, B, us_ind, vs_ind):
    pad = N_PAD - N
    ui = jnp.concatenate(
        [us_ind.astype(jnp.int32), jnp.zeros((pad,), jnp.int32)])
    vi = jnp.concatenate(
        [vs_ind.astype(jnp.int32), jnp.zeros((pad,), jnp.int32)])
    pu = (ui >= SPLIT).astype(jnp.int32)
    pv = (vi >= SPLIT).astype(jnp.int32)
    uih, vih = ui - pu * SPLIT, vi - pv * SPLIT
    pu, pv = pu.reshape(N_PAD, 1), pv.reshape(N_PAD, 1)

    U2 = _tc_repack(U.T)
    UG = _sc_gather(U2, uih)
    V2 = _tc_repack(V.T)
    VG = _sc_gather(V2, vih)

    B00 = B[:EMB, :EMB]
    bu = B[:EMB, EMB:]
    bv = B[EMB:, :EMB]
    c11 = B[EMB:, EMB:]
    out = _tc_bilinear(UG, VG, pu, pv, B00, bu, bv, c11)
    return out[:N]
